# baseline (speedup 1.0000x reference)
import functools

import jax
import jax.numpy as jnp
from jax import lax
from jax.experimental import pallas as pl
from jax.experimental.pallas import tpu as pltpu
from jax.experimental.pallas import tpu_sc as plsc

_NBUF = 4


@functools.lru_cache(maxsize=None)
def _build_gather(B0, S, V, D):
    info = plsc.get_sparse_core_info()
    nc, ns = info.num_cores, info.num_subcores
    nw = nc * ns
    n = B0 // nw
    gmax = (n - 2 * _NBUF + 1) // _NBUF
    main_end = _NBUF * (gmax + 1)
    assert gmax >= 1 and main_end <= n
    mesh = plsc.VectorSubcoreMesh(core_axis_name="c", subcore_axis_name="s")

    @functools.partial(
        pl.kernel,
        mesh=mesh,
        out_type=jax.ShapeDtypeStruct((B0, S, D), jnp.float32),
        scratch_types=[
            pltpu.VMEM((n, S), jnp.int32),
            [pltpu.VMEM((S, D), jnp.float32) for _ in range(_NBUF)],
            pltpu.SemaphoreType.DMA,
            pltpu.SemaphoreType.DMA,
        ],
        compiler_params=pltpu.CompilerParams(use_tc_tiling_on_sc=False),
    )
    def k(idx_hbm, table_hbm, out_hbm, idx_v, bufs, sem_g, sem_o):
        wid = lax.axis_index("s") * nc + lax.axis_index("c")
        base = wid * n
        pltpu.sync_copy(idx_hbm.at[pl.ds(base, n)], idx_v)

        def gather(j, buf):
            pltpu.async_copy(table_hbm.at[idx_v.at[j]], buf, sem_g)

        def put(j, buf):
            pltpu.async_copy(buf, out_hbm.at[base + j], sem_o)

        def wait_gather(buf):
            pltpu.make_async_copy(table_hbm.at[pl.ds(0, S)], buf, sem_g).wait()

        def wait_out(buf):
            pltpu.make_async_copy(buf, out_hbm.at[0], sem_o).wait()

        def step(j, b, first=False):
            wait_gather(bufs[b])
            put(j, bufs[b])
            nb = (b + _NBUF - 1) % _NBUF
            if not first:
                wait_out(bufs[nb])
            gather(j + _NBUF - 1, bufs[nb])

        for j in range(_NBUF - 1):
            gather(j, bufs[j])

        for j in range(_NBUF):
            step(j, j % _NBUF, first=(j == 0))

        def group(g, carry):
            for b in range(_NBUF):
                step(g * _NBUF + b, b)
            return carry

        lax.fori_loop(1, gmax + 1, group, 0)

        for j in range(main_end, n):
            b = j % _NBUF
            wait_gather(bufs[b])
            put(j, bufs[b])
            if j + _NBUF - 1 < n:
                nb = (b + _NBUF - 1) % _NBUF
                wait_out(bufs[nb])
                gather(j + _NBUF - 1, bufs[nb])

        for i in range(_NBUF):
            wait_out(bufs[i])

    return k


def kernel(inputs, embeddings):
    B0, S = inputs.shape
    V, D = embeddings.shape
    idx = inputs.astype(jnp.int32)
    return _build_gather(B0, S, V, D)(idx, embeddings)


# speedup vs baseline: 1.1208x; 1.1208x over previous
import functools

import jax
import jax.numpy as jnp
from jax import lax
from jax.experimental import pallas as pl
from jax.experimental.pallas import tpu as pltpu
from jax.experimental.pallas import tpu_sc as plsc

_NBUF = 4
_LANES = 128


@functools.lru_cache(maxsize=None)
def _build_gather(B0, S, V, D):
    info = plsc.get_sparse_core_info()
    nc, ns = info.num_cores, info.num_subcores
    nw = nc * ns
    n = B0 // nw
    gmax = (n - 2 * _NBUF + 1) // _NBUF
    main_end = _NBUF * (gmax + 1)
    assert gmax >= 1 and main_end <= n
    mesh = plsc.VectorSubcoreMesh(core_axis_name="c", subcore_axis_name="s")

    @functools.partial(
        pl.kernel,
        mesh=mesh,
        out_type=jax.ShapeDtypeStruct((B0, S, D), jnp.float32),
        scratch_types=[
            pltpu.VMEM((n, S), jnp.int32),
            [pltpu.VMEM((S, _LANES), jnp.float32) for _ in range(_NBUF)],
            [pltpu.VMEM((S, D), jnp.float32) for _ in range(_NBUF)],
            pltpu.SemaphoreType.DMA,
            pltpu.SemaphoreType.DMA,
        ],
    )
    def k(idx_hbm, table_hbm, out_hbm, idx_v, bufs, obufs, sem_g, sem_o):
        wid = lax.axis_index("s") * nc + lax.axis_index("c")
        base = wid * n
        pltpu.sync_copy(idx_hbm.at[pl.ds(base, n)], idx_v)

        def gather(j, buf):
            pltpu.async_copy(table_hbm.at[idx_v.at[j]], buf, sem_g)

        def put(j, obuf):
            pltpu.async_copy(obuf, out_hbm.at[base + j], sem_o)

        def wait_gather(j, buf):
            pltpu.make_async_copy(
                table_hbm.at[idx_v.at[j]], buf, sem_g
            ).wait()

        def wait_out(j, obuf):
            pltpu.make_async_copy(obuf, out_hbm.at[base + j], sem_o).wait()

        def vcopy(g, o):
            def body(s5, carry):
                for u in range(5):
                    s = s5 * 5 + u
                    for c in range(D // 16):
                        o[s, pl.ds(c * 16, 16)] = g[s, pl.ds(c * 16, 16)]
                return carry

            lax.fori_loop(0, S // 5, body, 0)

        def step(j, b, do_gather=True, do_retire=True):
            wait_gather(j, bufs[b])
            if do_gather:
                gather(j + _NBUF - 1, bufs[(b + _NBUF - 1) % _NBUF])
            if do_retire:
                wait_out(j - _NBUF, obufs[b])
            vcopy(bufs[b], obufs[b])
            put(j, obufs[b])

        for j in range(_NBUF - 1):
            gather(j, bufs[j])

        for j in range(_NBUF):
            step(j, j % _NBUF, do_retire=False)

        def group(g, carry):
            for b in range(_NBUF):
                step(g * _NBUF + b, b)
            return carry

        lax.fori_loop(1, gmax + 1, group, 0)

        for j in range(main_end, n):
            step(j, j % _NBUF, do_gather=(j + _NBUF - 1 < n))

        for j in range(n - _NBUF, n):
            wait_out(j, obufs[j % _NBUF])

    return k


def kernel(inputs, embeddings):
    B0, S = inputs.shape
    V, D = embeddings.shape
    idx = inputs.astype(jnp.int32)
    table = jnp.pad(embeddings, ((0, 0), (0, _LANES - D)))
    return _build_gather(B0, S, V, D)(idx, table)


# speedup vs baseline: 1.1252x; 1.0039x over previous
import functools

import jax
import jax.numpy as jnp
from jax import lax
from jax.experimental import pallas as pl
from jax.experimental.pallas import tpu as pltpu
from jax.experimental.pallas import tpu_sc as plsc

_NBUF = 4
_LANES = 128


@functools.lru_cache(maxsize=None)
def _build_gather(B0, S, V, D):
    info = plsc.get_sparse_core_info()
    nc, ns = info.num_cores, info.num_subcores
    nw = nc * ns
    n = B0 // nw
    gmax = (n - 2 * _NBUF + 1) // _NBUF
    main_end = _NBUF * (gmax + 1)
    assert gmax >= 1 and main_end <= n
    mesh = plsc.VectorSubcoreMesh(core_axis_name="c", subcore_axis_name="s")
    sp = (S + 7) // 8 * 8

    @functools.partial(
        pl.kernel,
        mesh=mesh,
        out_type=jax.ShapeDtypeStruct((B0, S, D), jnp.float32),
        scratch_types=[
            pltpu.VMEM((n * sp,), jnp.int32),
            [pltpu.VMEM((S, _LANES), jnp.float32) for _ in range(_NBUF)],
            [pltpu.VMEM((S, D), jnp.float32) for _ in range(_NBUF)],
            pltpu.SemaphoreType.DMA,
            pltpu.SemaphoreType.DMA,
        ],
    )
    def k(idx_hbm, table_hbm, out_hbm, idx_v, bufs, obufs, sem_g, sem_o):
        wid = lax.axis_index("s") * nc + lax.axis_index("c")
        base = wid * n
        pltpu.sync_copy(idx_hbm.at[pl.ds(base * sp, n * sp)], idx_v)

        def gather(j, buf):
            pltpu.async_copy(
                table_hbm.at[idx_v.at[pl.ds(j * sp, S)]], buf, sem_g
            )

        def put(j, obuf):
            pltpu.async_copy(obuf, out_hbm.at[base + j], sem_o)

        def wait_gather(j, buf):
            pltpu.make_async_copy(
                table_hbm.at[idx_v.at[pl.ds(j * sp, S)]], buf, sem_g
            ).wait()

        def wait_out(j, obuf):
            pltpu.make_async_copy(obuf, out_hbm.at[base + j], sem_o).wait()

        def vcopy(g, o):
            def body(s5, carry):
                for u in range(5):
                    s = s5 * 5 + u
                    for c in range(D // 16):
                        o[s, pl.ds(c * 16, 16)] = g[s, pl.ds(c * 16, 16)]
                return carry

            lax.fori_loop(0, S // 5, body, 0)

        def step(j, b, do_gather=True, do_retire=True):
            wait_gather(j, bufs[b])
            if do_gather:
                gather(j + _NBUF - 1, bufs[(b + _NBUF - 1) % _NBUF])
            if do_retire:
                wait_out(j - _NBUF, obufs[b])
            vcopy(bufs[b], obufs[b])
            put(j, obufs[b])

        for j in range(_NBUF - 1):
            gather(j, bufs[j])

        for j in range(_NBUF):
            step(j, j % _NBUF, do_retire=False)

        def group(g, carry):
            for b in range(_NBUF):
                step(g * _NBUF + b, b)
            return carry

        lax.fori_loop(1, gmax + 1, group, 0)

        for j in range(main_end, n):
            step(j, j % _NBUF, do_gather=(j + _NBUF - 1 < n))

        for j in range(n - _NBUF, n):
            wait_out(j, obufs[j % _NBUF])

    return k


def kernel(inputs, embeddings):
    B0, S = inputs.shape
    V, D = embeddings.shape
    sp = (S + 7) // 8 * 8
    idx = jnp.pad(inputs.astype(jnp.int32), ((0, 0), (0, sp - S))).reshape(-1)
    table = jnp.pad(embeddings, ((0, 0), (0, _LANES - D)))
    return _build_gather(B0, S, V, D)(idx, table)
